# baseline (speedup 1.0000x reference)
import jax
import jax.numpy as jnp
from jax.experimental import pallas as pl


def _copy_body(x_ref, o_ref):
    o_ref[...] = x_ref[...]


def _gcn(x, src, dst, dis, W, b, N):
    h = x @ W
    norm = dis[src] * dis[dst]
    out = jax.ops.segment_sum(h[src] * norm[:, None], dst, num_segments=N)
    out = out + h * (dis * dis)[:, None]
    return out + b


def _bn(x, g, be):
    mu = jnp.mean(x, axis=0)
    var = jnp.mean((x - mu) ** 2, axis=0)
    return g * (x - mu) / jnp.sqrt(var + 1e-5) + be


def _lrelu(x):
    return jnp.where(x >= 0, x, 0.01 * x)


def kernel(x, edge_index, params):
    x = pl.pallas_call(
        _copy_body,
        out_shape=jax.ShapeDtypeStruct(x.shape, x.dtype),
    )(x)
    src = edge_index[0]
    dst = edge_index[1]
    N = x.shape[0]
    deg = 1.0 + jax.ops.segment_sum(jnp.ones_like(dst, dtype=jnp.float32), dst, num_segments=N)
    dis = jax.lax.rsqrt(deg)
    h = x
    for i in range(12):
        h = _lrelu(_bn(_gcn(h, src, dst, dis, params["W"][i], params["b"][i], N), params["g"][i], params["be"][i]))
    h = _lrelu(h @ params["lW1"] + params["lb1"])
    return h @ params["lW2"] + params["lb2"]


# speedup vs baseline: 1.8794x; 1.8794x over previous
import functools

import jax
import jax.numpy as jnp
from jax import lax
from jax.experimental import pallas as pl
from jax.experimental.pallas import tpu as pltpu, tpu_sc as plsc

N = 50000
RNG = 1568
NBIN = 32
NP = NBIN * RNG
NT = 32
EP = 800256
EPT = EP // NT
CAP = ((EPT + 127) // 128 + 1) * 128
SENT = N
BM = 1024
NBLK_ROWS = NP // BM

_SC_PARAMS = pltpu.CompilerParams(use_tc_tiling_on_sc=False,
                                  needs_layout_passes=False)
def _mesh():
    return plsc.VectorSubcoreMesh(core_axis_name="c", subcore_axis_name="s")

_BPB = {16: 16, 32: 16, 64: 8, 128: 4, 256: 2, 512: 1}


def _widx(cid, sid):
    return cid * 16 + sid


def _pre_body(src_h, dst_h, bsrc_h, bdstl_h, cnts_h,
              src_v, dst_v, osrc_v, odstl_v, cnt_v):
    wid = _widx(lax.axis_index("c"), lax.axis_index("s"))
    base_e = pl.multiple_of(wid * EPT, 8)
    pltpu.sync_copy(src_h.at[pl.ds(base_e, EPT)], src_v)
    pltpu.sync_copy(dst_h.at[pl.ds(base_e, EPT)], dst_v)

    nchunk = EPT // 16

    def h_body(i, cnts):
        d16 = dst_v[pl.ds(i * 16, 16)]
        bid = d16 // RNG
        return tuple(cnts[b] + jnp.sum((bid == b).astype(jnp.int32))
                     for b in range(NBIN))

    cnts = lax.fori_loop(0, nchunk, h_body, (jnp.int32(0),) * NBIN)

    pads = [((c + 127) // 128) * 128 for c in cnts]
    bases = []
    acc = jnp.int32(0)
    for b in range(NBIN):
        bases.append(acc)
        acc = acc + pads[b]

    def c_body(i, curs):
        d16 = dst_v[pl.ds(i * 16, 16)]
        s16 = src_v[pl.ds(i * 16, 16)]
        bid = d16 // RNG
        dstl = d16 - bid * RNG
        new = []
        for b in range(NBIN):
            m = bid == b
            plsc.store_compressed(osrc_v.at[pl.ds(curs[b], 16)], s16, mask=m)
            plsc.store_compressed(odstl_v.at[pl.ds(curs[b], 16)], dstl, mask=m)
            new.append(curs[b] + jnp.sum(m.astype(jnp.int32)))
        return tuple(new)

    ends = lax.fori_loop(0, nchunk, c_body, tuple(bases))

    lane = lax.iota(jnp.int32, 16)
    sent_s = jnp.full((16,), SENT, jnp.int32)
    sent_d = jnp.zeros((16,), jnp.int32)
    for b in range(NBIN):
        room = bases[b] + pads[b] - ends[b]
        for k in range(8):
            m = (lane + k * 16) < room
            pos = ends[b] + lane + k * 16
            plsc.store_scatter(osrc_v, [pos], sent_s, mask=m)
            plsc.store_scatter(odstl_v, [pos], sent_d, mask=m)

    for b in range(NBIN):
        hbase = (b * NT + wid) * CAP
        nch = pads[b] // 128

        def d_body(j, _, b=b, hbase=hbase):
            voff = pl.multiple_of(bases[b] + j * 128, 128)
            hoff = pl.multiple_of(hbase + j * 128, 128)
            pltpu.sync_copy(osrc_v.at[pl.ds(voff, 128)],
                            bsrc_h.at[pl.ds(hoff, 128)])
            pltpu.sync_copy(odstl_v.at[pl.ds(voff, 128)],
                            bdstl_h.at[pl.ds(hoff, 128)])
            return 0

        lax.fori_loop(0, nch, d_body, 0)
        plsc.store_scatter(cnt_v, [jnp.full((16,), b, jnp.int32)],
                           jnp.full((16,), pads[b], jnp.int32),
                           mask=lane == 0)
    pltpu.sync_copy(cnt_v, cnts_h.at[pl.ds(pl.multiple_of(wid * NBIN, 32), NBIN)])


@functools.lru_cache(maxsize=None)
def _make_preprocess():
    @functools.partial(
        pl.kernel, mesh=_mesh(), compiler_params=_SC_PARAMS,
        name="edge_binning",
        out_type=(jax.ShapeDtypeStruct((NBIN * NT * CAP,), jnp.int32),
                  jax.ShapeDtypeStruct((NBIN * NT * CAP,), jnp.int32),
                  jax.ShapeDtypeStruct((NT * NBIN,), jnp.int32)),
        scratch_types=[
            pltpu.VMEM((EPT,), jnp.int32),
            pltpu.VMEM((EPT,), jnp.int32),
            pltpu.VMEM((EPT + NBIN * 128 + 144,), jnp.int32),
            pltpu.VMEM((EPT + NBIN * 128 + 144,), jnp.int32),
            pltpu.VMEM((NBIN,), jnp.int32),
        ],
    )
    def _preprocess(*args):
        _pre_body(*args)

    return _preprocess


def _deg_body(bsrc_h, bdstl_h, cnts_h, deg_h, idx_v, src_v, acc_v, cnt_v):
    wid = _widx(lax.axis_index("c"), lax.axis_index("s"))
    pltpu.sync_copy(cnts_h, cnt_v)
    for k in range(RNG // 16):
        acc_v[pl.ds(k * 16, 16)] = jnp.zeros((16,), jnp.float32)
    for pt in range(NT):
        c = plsc.load_gather(
            cnt_v, [jnp.full((16,), pt * NBIN + wid, jnp.int32)])[0]
        hbase = (wid * NT + pt) * CAP

        def ch_body(j, _, hbase=hbase):
            hoff = pl.multiple_of(hbase + j * 128, 128)
            pltpu.sync_copy(bdstl_h.at[pl.ds(hoff, 128)], idx_v)
            pltpu.sync_copy(bsrc_h.at[pl.ds(hoff, 128)], src_v)
            for k in range(8):
                i16 = idx_v[pl.ds(k * 16, 16)]
                s16 = src_v[pl.ds(k * 16, 16)]
                ones = jnp.where(s16 == SENT, 0.0, 1.0).astype(jnp.float32)
                plsc.addupdate_scatter(acc_v, [i16], ones)
            return 0

        lax.fori_loop(0, c // 128, ch_body, 0)
    pltpu.sync_copy(acc_v, deg_h.at[pl.ds(pl.multiple_of(wid * RNG, 32), RNG)])


@functools.lru_cache(maxsize=None)
def _make_degrees():
    @functools.partial(
        pl.kernel, mesh=_mesh(), compiler_params=_SC_PARAMS,
        name="degree_histogram",
        out_type=jax.ShapeDtypeStruct((NP,), jnp.float32),
        scratch_types=[
            pltpu.VMEM((128,), jnp.int32),
            pltpu.VMEM((128,), jnp.int32),
            pltpu.VMEM((RNG,), jnp.float32),
            pltpu.VMEM((NT * NBIN,), jnp.int32),
        ],
    )
    def _degrees(*args):
        _deg_body(*args)

    return _degrees


def _seg_body(F, bpb, zr, h_h, bsrc_h, bdstl_h, cnts_h, z_h, s_h,
              idx_v, dstl_v, rows_v, zero_v, cnt_v, acc_sh, sem):
    cid = lax.axis_index("c")
    sid = lax.axis_index("s")
    nblk = NBIN // bpb
    blk_per_sc = nblk // 2
    rows_blk = bpb * RNG
    share = rows_blk // 16
    pltpu.sync_copy(cnts_h, cnt_v)
    pltpu.sync_copy(z_h, zero_v)

    for kb in range(blk_per_sc):
        b = cid * blk_per_sc + kb
        row0 = b * rows_blk
        my0 = sid * share
        for z in range(share // zr):
            pltpu.sync_copy(zero_v, acc_sh.at[pl.ds(my0 + z * zr, zr), :])
        plsc.subcore_barrier()
        for j in range(2 * bpb):
            rr = sid * (2 * bpb) + j
            boff = rr // 32
            pt = rr % 32
            bin_ = b * bpb + boff
            c = plsc.load_gather(
                cnt_v, [jnp.full((16,), pt * NBIN, jnp.int32) + bin_])[0]
            hbase = (bin_ * NT + pt) * CAP

            def a_body(j2, _, boff=boff, hbase=hbase):
                hoff = pl.multiple_of(hbase + j2 * 128, 128)
                pltpu.sync_copy(bsrc_h.at[pl.ds(hoff, 128)], idx_v)
                pltpu.sync_copy(bdstl_h.at[pl.ds(hoff, 128)], dstl_v)
                for k in range(8):
                    dstl_v[pl.ds(k * 16, 16)] = (
                        dstl_v[pl.ds(k * 16, 16)] + boff * RNG)
                pltpu.async_copy(h_h.at[idx_v], rows_v, sem).wait()
                pltpu.sync_copy(rows_v, acc_sh.at[dstl_v], add=True)
                return 0

            lax.fori_loop(0, c // 128, a_body, 0)
        plsc.subcore_barrier()
        for z in range(share // 49):
            r0 = my0 + z * 49
            pltpu.sync_copy(acc_sh.at[pl.ds(r0, 49), :],
                            rows_v.at[pl.ds(0, 49), :])
            pltpu.sync_copy(rows_v.at[pl.ds(0, 49), :],
                            s_h.at[pl.ds(row0 + r0, 49), :])
        plsc.subcore_barrier()


@functools.lru_cache(maxsize=None)
def _make_segsum(F):
    bpb = _BPB[F]
    zr = 7 if F >= 512 else 49

    @functools.partial(
        pl.kernel, mesh=_mesh(), compiler_params=_SC_PARAMS,
        name=f"segsum_f{F}",
        out_type=jax.ShapeDtypeStruct((NP, F), jnp.float32),
        scratch_types=[
            pltpu.VMEM((128,), jnp.int32),
            pltpu.VMEM((128,), jnp.int32),
            pltpu.VMEM((128, F), jnp.float32),
            pltpu.VMEM((zr, F), jnp.float32),
            pltpu.VMEM((NT * NBIN,), jnp.int32),
            pltpu.MemorySpace.VMEM_SHARED((bpb * RNG, F), jnp.float32),
            pltpu.SemaphoreType.DMA,
        ],
    )
    def seg(*args):
        _seg_body(F, bpb, zr, *args)

    return seg


def _lrelu(x):
    return jnp.where(x >= 0, x, 0.01 * x)


def _dot(a, b):
    return lax.dot_general(a, b, (((1,), (0,)), ((), ())),
                           preferred_element_type=jnp.float32)


def _dis_body(cnt_ref, dis_ref):
    g = pl.program_id(0)
    rows = lax.broadcasted_iota(jnp.int32, (BM, 1), 0) + g * BM
    dis_ref[...] = jnp.where(rows < N,
                             lax.rsqrt(1.0 + cnt_ref[...]), 0.0)


def _dis_kernel(cnt2):
    return pl.pallas_call(
        _dis_body,
        grid=(NBLK_ROWS,),
        in_specs=[pl.BlockSpec((BM, 1), lambda g: (g, 0))],
        out_specs=pl.BlockSpec((BM, 1), lambda g: (g, 0)),
        out_shape=jax.ShapeDtypeStruct((NP, 1), jnp.float32),
    )(cnt2)


def _k10_body(x_ref, W_ref, dis_ref, out_ref):
    out_ref[...] = _dot(x_ref[...], W_ref[...]) * dis_ref[...]


def _k1_first(xp, W0, dis):
    Fi, Fo = W0.shape
    return pl.pallas_call(
        _k10_body,
        grid=(NBLK_ROWS,),
        in_specs=[pl.BlockSpec((BM, Fi), lambda g: (g, 0)),
                  pl.BlockSpec((Fi, Fo), lambda g: (0, 0)),
                  pl.BlockSpec((BM, 1), lambda g: (g, 0))],
        out_specs=pl.BlockSpec((BM, Fo), lambda g: (g, 0)),
        out_shape=jax.ShapeDtypeStruct((NP, Fo), jnp.float32),
    )(xp, W0, dis)


def _bn(pre, sums, vsums, gam, bet):
    mu = sums[0, :] / N
    var = vsums[0, :] / N
    sd = jnp.sqrt(var + 1e-5)
    return gam[0, :][None, :] * (pre - mu[None, :]) / sd[None, :] \
        + bet[0, :][None, :]


def _k1_body(pre_ref, sums_ref, vsums_ref, gam_ref, bet_ref, W_ref, dis_ref,
             out_ref):
    t = _lrelu(_bn(pre_ref[...], sums_ref, vsums_ref, gam_ref, bet_ref))
    out_ref[...] = _dot(t, W_ref[...]) * dis_ref[...]


def _k1_mid(pre, sums, vsums, gam, bet, W, dis):
    Fi, Fo = W.shape
    return pl.pallas_call(
        _k1_body,
        grid=(NBLK_ROWS,),
        in_specs=[pl.BlockSpec((BM, Fi), lambda g: (g, 0)),
                  pl.BlockSpec((8, Fi), lambda g: (0, 0)),
                  pl.BlockSpec((8, Fi), lambda g: (0, 0)),
                  pl.BlockSpec((1, Fi), lambda g: (0, 0)),
                  pl.BlockSpec((1, Fi), lambda g: (0, 0)),
                  pl.BlockSpec((Fi, Fo), lambda g: (0, 0)),
                  pl.BlockSpec((BM, 1), lambda g: (g, 0))],
        out_specs=pl.BlockSpec((BM, Fo), lambda g: (g, 0)),
        out_shape=jax.ShapeDtypeStruct((NP, Fo), jnp.float32),
    )(pre, sums, vsums, gam, bet, W, dis)


def _k3_body(s_ref, h_ref, dis_ref, b_ref, pre_ref, sums_ref):
    g = pl.program_id(0)
    pre = (s_ref[...] * dis_ref[...] + h_ref[...] * dis_ref[...]
           ) + b_ref[0, :][None, :]
    pre_ref[...] = pre
    rows = lax.broadcasted_iota(jnp.int32, (BM, 1), 0) + g * BM
    m = (rows < N).astype(jnp.float32)
    s1 = jnp.sum(pre * m, axis=0)
    Fo = s1.shape[0]
    blk = jnp.concatenate([s1[None, :],
                           jnp.zeros((7, Fo), jnp.float32)], axis=0)

    @pl.when(g == 0)
    def _():
        sums_ref[...] = jnp.zeros_like(sums_ref)

    sums_ref[...] += blk


def _kvar_body(pre_ref, sums_ref, vsums_ref):
    g = pl.program_id(0)
    mu = sums_ref[0, :] / N
    rows = lax.broadcasted_iota(jnp.int32, (BM, 1), 0) + g * BM
    m = (rows < N).astype(jnp.float32)
    d = pre_ref[...] - mu[None, :]
    s2 = jnp.sum(d * d * m, axis=0)
    Fo = s2.shape[0]
    blk = jnp.concatenate([s2[None, :],
                           jnp.zeros((7, Fo), jnp.float32)], axis=0)

    @pl.when(g == 0)
    def _():
        vsums_ref[...] = jnp.zeros_like(vsums_ref)

    vsums_ref[...] += blk


def _kvar(pre, sums):
    Fo = pre.shape[1]
    return pl.pallas_call(
        _kvar_body,
        grid=(NBLK_ROWS,),
        in_specs=[pl.BlockSpec((BM, Fo), lambda g: (g, 0)),
                  pl.BlockSpec((8, Fo), lambda g: (0, 0))],
        out_specs=pl.BlockSpec((8, Fo), lambda g: (0, 0)),
        out_shape=jax.ShapeDtypeStruct((8, Fo), jnp.float32),
    )(pre, sums)


def _k3(s, h, dis, b):
    Fo = s.shape[1]
    return pl.pallas_call(
        _k3_body,
        grid=(NBLK_ROWS,),
        in_specs=[pl.BlockSpec((BM, Fo), lambda g: (g, 0)),
                  pl.BlockSpec((BM, Fo), lambda g: (g, 0)),
                  pl.BlockSpec((BM, 1), lambda g: (g, 0)),
                  pl.BlockSpec((1, Fo), lambda g: (0, 0))],
        out_specs=[pl.BlockSpec((BM, Fo), lambda g: (g, 0)),
                   pl.BlockSpec((8, Fo), lambda g: (0, 0))],
        out_shape=[jax.ShapeDtypeStruct((NP, Fo), jnp.float32),
                   jax.ShapeDtypeStruct((8, Fo), jnp.float32)],
    )(s, h, dis, b)


def _head_body(pre_ref, sums_ref, vsums_ref, gam_ref, bet_ref,
               w1_ref, b1_ref, w2_ref, b2_ref, out_ref):
    t = _lrelu(_bn(pre_ref[...], sums_ref, vsums_ref, gam_ref, bet_ref))
    u = _lrelu(_dot(t, w1_ref[...]) + b1_ref[0, :][None, :])
    out_ref[...] = _dot(u, w2_ref[...]) + b2_ref[0, :][None, :]


def _head(pre, sums, vsums, gam, bet, w1, b1, w2, b2):
    Fi = pre.shape[1]
    F1 = w1.shape[1]
    F2 = w2.shape[1]
    return pl.pallas_call(
        _head_body,
        grid=(NBLK_ROWS,),
        in_specs=[pl.BlockSpec((BM, Fi), lambda g: (g, 0)),
                  pl.BlockSpec((8, Fi), lambda g: (0, 0)),
                  pl.BlockSpec((8, Fi), lambda g: (0, 0)),
                  pl.BlockSpec((1, Fi), lambda g: (0, 0)),
                  pl.BlockSpec((1, Fi), lambda g: (0, 0)),
                  pl.BlockSpec((Fi, F1), lambda g: (0, 0)),
                  pl.BlockSpec((1, F1), lambda g: (0, 0)),
                  pl.BlockSpec((F1, F2), lambda g: (0, 0)),
                  pl.BlockSpec((1, F2), lambda g: (0, 0))],
        out_specs=pl.BlockSpec((BM, F2), lambda g: (g, 0)),
        out_shape=jax.ShapeDtypeStruct((NP, F2), jnp.float32),
    )(pre, sums, vsums, gam, bet, w1, b1, w2, b2)


def kernel(x, edge_index, params):
    E = edge_index.shape[1]
    pad = EP - E
    sent = jnp.concatenate(
        [jnp.full((1, pad), SENT, jnp.int32), jnp.zeros((1, pad), jnp.int32)],
        axis=0)
    eix = jnp.concatenate([edge_index.astype(jnp.int32), sent], axis=1)
    src = eix[0]
    dst = eix[1]

    bsrc, bdstl, cnts = _make_preprocess()(src, dst)
    cnt = _make_degrees()(bsrc, bdstl, cnts)
    dis = _dis_kernel(cnt.reshape(NP, 1))

    xp = jnp.pad(x, ((0, NP - x.shape[0]), (0, 0)))
    h = _k1_first(xp, params["W"][0], dis)
    pre = None
    sums = None
    for i in range(12):
        Fo = h.shape[1]
        zF = jnp.zeros((7 if Fo >= 512 else 49, Fo), jnp.float32)
        s = _make_segsum(Fo)(h, bsrc, bdstl, cnts, zF)
        pre, sums = _k3(s, h, dis, params["b"][i].reshape(1, -1))
        vsums = _kvar(pre, sums)
        if i < 11:
            h = _k1_mid(pre, sums, vsums,
                        params["g"][i].reshape(1, -1),
                        params["be"][i].reshape(1, -1),
                        params["W"][i + 1], dis)
    out = _head(pre, sums, vsums,
                params["g"][11].reshape(1, -1),
                params["be"][11].reshape(1, -1),
                params["lW1"], params["lb1"].reshape(1, -1),
                params["lW2"], params["lb2"].reshape(1, -1))
    return out[:N]


# speedup vs baseline: 1.9091x; 1.0158x over previous
import functools

import jax
import jax.numpy as jnp
from jax import lax
from jax.experimental import pallas as pl
from jax.experimental.pallas import tpu as pltpu, tpu_sc as plsc

N = 50000
RNG = 1568
NBIN = 32
NP = NBIN * RNG
NT = 32
EP = 800256
EPT = EP // NT
CAP = ((EPT + 127) // 128 + 1) * 128
SENT = N
BM = 1024
NBLK_ROWS = NP // BM

_SC_PARAMS = pltpu.CompilerParams(use_tc_tiling_on_sc=False,
                                  needs_layout_passes=False)
def _mesh():
    return plsc.VectorSubcoreMesh(core_axis_name="c", subcore_axis_name="s")

_BPB = {16: 16, 32: 16, 64: 8, 128: 4, 256: 2, 512: 1}


def _widx(cid, sid):
    return cid * 16 + sid


def _pre_body(src_h, dst_h, bsrc_h, bdstl_h, cnts_h,
              src_v, dst_v, osrc_v, odstl_v, cnt_v):
    wid = _widx(lax.axis_index("c"), lax.axis_index("s"))
    base_e = pl.multiple_of(wid * EPT, 8)
    pltpu.sync_copy(src_h.at[pl.ds(base_e, EPT)], src_v)
    pltpu.sync_copy(dst_h.at[pl.ds(base_e, EPT)], dst_v)

    nchunk = EPT // 16

    def h_body(i, cnts):
        d16 = dst_v[pl.ds(i * 16, 16)]
        bid = d16 // RNG
        return tuple(cnts[b] + jnp.sum((bid == b).astype(jnp.int32))
                     for b in range(NBIN))

    cnts = lax.fori_loop(0, nchunk, h_body, (jnp.int32(0),) * NBIN)

    pads = [((c + 127) // 128) * 128 for c in cnts]
    bases = []
    acc = jnp.int32(0)
    for b in range(NBIN):
        bases.append(acc)
        acc = acc + pads[b]

    def c_body(i, curs):
        d16 = dst_v[pl.ds(i * 16, 16)]
        s16 = src_v[pl.ds(i * 16, 16)]
        bid = d16 // RNG
        dstl = d16 - bid * RNG
        new = []
        for b in range(NBIN):
            m = bid == b
            plsc.store_compressed(osrc_v.at[pl.ds(curs[b], 16)], s16, mask=m)
            plsc.store_compressed(odstl_v.at[pl.ds(curs[b], 16)], dstl, mask=m)
            new.append(curs[b] + jnp.sum(m.astype(jnp.int32)))
        return tuple(new)

    ends = lax.fori_loop(0, nchunk, c_body, tuple(bases))

    lane = lax.iota(jnp.int32, 16)
    sent_s = jnp.full((16,), SENT, jnp.int32)
    sent_d = jnp.zeros((16,), jnp.int32)
    for b in range(NBIN):
        room = bases[b] + pads[b] - ends[b]
        for k in range(8):
            m = (lane + k * 16) < room
            pos = ends[b] + lane + k * 16
            plsc.store_scatter(osrc_v, [pos], sent_s, mask=m)
            plsc.store_scatter(odstl_v, [pos], sent_d, mask=m)

    for b in range(NBIN):
        hbase = (b * NT + wid) * CAP
        nch = pads[b] // 128

        def d_body(j, _, b=b, hbase=hbase):
            voff = pl.multiple_of(bases[b] + j * 128, 128)
            hoff = pl.multiple_of(hbase + j * 128, 128)
            pltpu.sync_copy(osrc_v.at[pl.ds(voff, 128)],
                            bsrc_h.at[pl.ds(hoff, 128)])
            pltpu.sync_copy(odstl_v.at[pl.ds(voff, 128)],
                            bdstl_h.at[pl.ds(hoff, 128)])
            return 0

        lax.fori_loop(0, nch, d_body, 0)
        plsc.store_scatter(cnt_v, [jnp.full((16,), b, jnp.int32)],
                           jnp.full((16,), pads[b], jnp.int32),
                           mask=lane == 0)
    pltpu.sync_copy(cnt_v, cnts_h.at[pl.ds(pl.multiple_of(wid * NBIN, 32), NBIN)])


@functools.lru_cache(maxsize=None)
def _make_preprocess():
    @functools.partial(
        pl.kernel, mesh=_mesh(), compiler_params=_SC_PARAMS,
        name="edge_binning",
        out_type=(jax.ShapeDtypeStruct((NBIN * NT * CAP,), jnp.int32),
                  jax.ShapeDtypeStruct((NBIN * NT * CAP,), jnp.int32),
                  jax.ShapeDtypeStruct((NT * NBIN,), jnp.int32)),
        scratch_types=[
            pltpu.VMEM((EPT,), jnp.int32),
            pltpu.VMEM((EPT,), jnp.int32),
            pltpu.VMEM((EPT + NBIN * 128 + 144,), jnp.int32),
            pltpu.VMEM((EPT + NBIN * 128 + 144,), jnp.int32),
            pltpu.VMEM((NBIN,), jnp.int32),
        ],
    )
    def _preprocess(*args):
        _pre_body(*args)

    return _preprocess


def _deg_body(bsrc_h, bdstl_h, cnts_h, deg_h, idx_v, src_v, acc_v, cnt_v):
    wid = _widx(lax.axis_index("c"), lax.axis_index("s"))
    pltpu.sync_copy(cnts_h, cnt_v)
    for k in range(RNG // 16):
        acc_v[pl.ds(k * 16, 16)] = jnp.zeros((16,), jnp.float32)
    for pt in range(NT):
        c = plsc.load_gather(
            cnt_v, [jnp.full((16,), pt * NBIN + wid, jnp.int32)])[0]
        hbase = (wid * NT + pt) * CAP

        def ch_body(j, _, hbase=hbase):
            hoff = pl.multiple_of(hbase + j * 128, 128)
            pltpu.sync_copy(bdstl_h.at[pl.ds(hoff, 128)], idx_v)
            pltpu.sync_copy(bsrc_h.at[pl.ds(hoff, 128)], src_v)
            for k in range(8):
                i16 = idx_v[pl.ds(k * 16, 16)]
                s16 = src_v[pl.ds(k * 16, 16)]
                ones = jnp.where(s16 == SENT, 0.0, 1.0).astype(jnp.float32)
                plsc.addupdate_scatter(acc_v, [i16], ones)
            return 0

        lax.fori_loop(0, c // 128, ch_body, 0)
    pltpu.sync_copy(acc_v, deg_h.at[pl.ds(pl.multiple_of(wid * RNG, 32), RNG)])


@functools.lru_cache(maxsize=None)
def _make_degrees():
    @functools.partial(
        pl.kernel, mesh=_mesh(), compiler_params=_SC_PARAMS,
        name="degree_histogram",
        out_type=jax.ShapeDtypeStruct((NP,), jnp.float32),
        scratch_types=[
            pltpu.VMEM((128,), jnp.int32),
            pltpu.VMEM((128,), jnp.int32),
            pltpu.VMEM((RNG,), jnp.float32),
            pltpu.VMEM((NT * NBIN,), jnp.int32),
        ],
    )
    def _degrees(*args):
        _deg_body(*args)

    return _degrees


def _seg_body(F, bpb, zr, h_h, bsrc_h, bdstl_h, cnts_h, z_h, s_h,
              idx_v, dstl_v, rows_v, zero_v, cnt_v, acc_sh, sem):
    cid = lax.axis_index("c")
    sid = lax.axis_index("s")
    nblk = NBIN // bpb
    blk_per_sc = nblk // 2
    rows_blk = bpb * RNG
    share = rows_blk // 16
    pltpu.sync_copy(cnts_h, cnt_v)
    pltpu.sync_copy(z_h, zero_v)

    for kb in range(blk_per_sc):
        b = cid * blk_per_sc + kb
        row0 = b * rows_blk
        my0 = sid * share
        for z in range(share // zr):
            pltpu.sync_copy(zero_v, acc_sh.at[pl.ds(my0 + z * zr, zr), :])
        plsc.subcore_barrier()
        ch = 64 if F >= 512 else 128
        for j in range(2 * bpb):
            rr = sid * (2 * bpb) + j
            boff = rr // 32
            pt = rr % 32
            bin_ = b * bpb + boff
            c = plsc.load_gather(
                cnt_v, [jnp.full((16,), pt * NBIN, jnp.int32) + bin_])[0]
            hbase = (bin_ * NT + pt) * CAP
            nch = c // ch

            def prefetch(j2, pbuf, boff=boff, hbase=hbase, ch=ch):
                hoff = pl.multiple_of(hbase + j2 * ch, ch)
                pltpu.sync_copy(bsrc_h.at[pl.ds(hoff, ch)], idx_v.at[pbuf])
                pltpu.sync_copy(bdstl_h.at[pl.ds(hoff, ch)], dstl_v.at[pbuf])
                for k in range(ch // 16):
                    dstl_v[pbuf, pl.ds(k * 16, 16)] = (
                        dstl_v[pbuf, pl.ds(k * 16, 16)] + boff * RNG)
                pltpu.async_copy(h_h.at[idx_v.at[pbuf]], rows_v.at[pbuf],
                                 sem)

            @pl.when(nch > 0)
            def _():
                prefetch(jnp.int32(0), jnp.int32(0))

            def a_body(j2, _, ch=ch):
                pbuf = jnp.bitwise_and(j2, 1)

                @pl.when(j2 + 1 < nch)
                def _():
                    prefetch(j2 + 1, 1 - pbuf)

                pltpu.make_async_copy(h_h.at[idx_v.at[pbuf]],
                                      rows_v.at[pbuf], sem).wait()
                pltpu.sync_copy(rows_v.at[pbuf], acc_sh.at[dstl_v.at[pbuf]],
                                add=True)
                return 0

            lax.fori_loop(0, nch, a_body, 0)
        plsc.subcore_barrier()
        for z in range(share // 49):
            r0 = my0 + z * 49
            pltpu.sync_copy(acc_sh.at[pl.ds(r0, 49), :],
                            rows_v.at[0, pl.ds(0, 49), :])
            pltpu.sync_copy(rows_v.at[0, pl.ds(0, 49), :],
                            s_h.at[pl.ds(row0 + r0, 49), :])
        plsc.subcore_barrier()


@functools.lru_cache(maxsize=None)
def _make_segsum(F):
    bpb = _BPB[F]
    zr = 7 if F >= 512 else 49

    @functools.partial(
        pl.kernel, mesh=_mesh(), compiler_params=_SC_PARAMS,
        name=f"segsum_f{F}",
        out_type=jax.ShapeDtypeStruct((NP, F), jnp.float32),
        scratch_types=[
            pltpu.VMEM((2, 64 if F >= 512 else 128), jnp.int32),
            pltpu.VMEM((2, 64 if F >= 512 else 128), jnp.int32),
            pltpu.VMEM((2, 64 if F >= 512 else 128, F), jnp.float32),
            pltpu.VMEM((zr, F), jnp.float32),
            pltpu.VMEM((NT * NBIN,), jnp.int32),
            pltpu.MemorySpace.VMEM_SHARED((bpb * RNG, F), jnp.float32),
            pltpu.SemaphoreType.DMA,
        ],
    )
    def seg(*args):
        _seg_body(F, bpb, zr, *args)

    return seg


def _lrelu(x):
    return jnp.where(x >= 0, x, 0.01 * x)


def _dot(a, b):
    return lax.dot_general(a, b, (((1,), (0,)), ((), ())),
                           preferred_element_type=jnp.float32)


def _dis_body(cnt_ref, dis_ref):
    g = pl.program_id(0)
    rows = lax.broadcasted_iota(jnp.int32, (BM, 1), 0) + g * BM
    dis_ref[...] = jnp.where(rows < N,
                             lax.rsqrt(1.0 + cnt_ref[...]), 0.0)


def _dis_kernel(cnt2):
    return pl.pallas_call(
        _dis_body,
        grid=(NBLK_ROWS,),
        in_specs=[pl.BlockSpec((BM, 1), lambda g: (g, 0))],
        out_specs=pl.BlockSpec((BM, 1), lambda g: (g, 0)),
        out_shape=jax.ShapeDtypeStruct((NP, 1), jnp.float32),
    )(cnt2)


def _k10_body(x_ref, W_ref, dis_ref, out_ref):
    out_ref[...] = _dot(x_ref[...], W_ref[...]) * dis_ref[...]


def _k1_first(xp, W0, dis):
    Fi, Fo = W0.shape
    return pl.pallas_call(
        _k10_body,
        grid=(NBLK_ROWS,),
        in_specs=[pl.BlockSpec((BM, Fi), lambda g: (g, 0)),
                  pl.BlockSpec((Fi, Fo), lambda g: (0, 0)),
                  pl.BlockSpec((BM, 1), lambda g: (g, 0))],
        out_specs=pl.BlockSpec((BM, Fo), lambda g: (g, 0)),
        out_shape=jax.ShapeDtypeStruct((NP, Fo), jnp.float32),
    )(xp, W0, dis)


def _bn(pre, sums, vsums, gam, bet):
    mu = sums[0, :] / N
    var = vsums[0, :] / N
    sd = jnp.sqrt(var + 1e-5)
    return gam[0, :][None, :] * (pre - mu[None, :]) / sd[None, :] \
        + bet[0, :][None, :]


def _k1_body(pre_ref, sums_ref, vsums_ref, gam_ref, bet_ref, W_ref, dis_ref,
             out_ref):
    t = _lrelu(_bn(pre_ref[...], sums_ref, vsums_ref, gam_ref, bet_ref))
    out_ref[...] = _dot(t, W_ref[...]) * dis_ref[...]


def _k1_mid(pre, sums, vsums, gam, bet, W, dis):
    Fi, Fo = W.shape
    return pl.pallas_call(
        _k1_body,
        grid=(NBLK_ROWS,),
        in_specs=[pl.BlockSpec((BM, Fi), lambda g: (g, 0)),
                  pl.BlockSpec((8, Fi), lambda g: (0, 0)),
                  pl.BlockSpec((8, Fi), lambda g: (0, 0)),
                  pl.BlockSpec((1, Fi), lambda g: (0, 0)),
                  pl.BlockSpec((1, Fi), lambda g: (0, 0)),
                  pl.BlockSpec((Fi, Fo), lambda g: (0, 0)),
                  pl.BlockSpec((BM, 1), lambda g: (g, 0))],
        out_specs=pl.BlockSpec((BM, Fo), lambda g: (g, 0)),
        out_shape=jax.ShapeDtypeStruct((NP, Fo), jnp.float32),
    )(pre, sums, vsums, gam, bet, W, dis)


def _k3_body(s_ref, h_ref, dis_ref, b_ref, pre_ref, sums_ref):
    g = pl.program_id(0)
    pre = (s_ref[...] * dis_ref[...] + h_ref[...] * dis_ref[...]
           ) + b_ref[0, :][None, :]
    pre_ref[...] = pre
    rows = lax.broadcasted_iota(jnp.int32, (BM, 1), 0) + g * BM
    m = (rows < N).astype(jnp.float32)
    s1 = jnp.sum(pre * m, axis=0)
    Fo = s1.shape[0]
    blk = jnp.concatenate([s1[None, :],
                           jnp.zeros((7, Fo), jnp.float32)], axis=0)

    @pl.when(g == 0)
    def _():
        sums_ref[...] = jnp.zeros_like(sums_ref)

    sums_ref[...] += blk


def _kvar_body(pre_ref, sums_ref, vsums_ref):
    g = pl.program_id(0)
    mu = sums_ref[0, :] / N
    rows = lax.broadcasted_iota(jnp.int32, (BM, 1), 0) + g * BM
    m = (rows < N).astype(jnp.float32)
    d = pre_ref[...] - mu[None, :]
    s2 = jnp.sum(d * d * m, axis=0)
    Fo = s2.shape[0]
    blk = jnp.concatenate([s2[None, :],
                           jnp.zeros((7, Fo), jnp.float32)], axis=0)

    @pl.when(g == 0)
    def _():
        vsums_ref[...] = jnp.zeros_like(vsums_ref)

    vsums_ref[...] += blk


def _kvar(pre, sums):
    Fo = pre.shape[1]
    return pl.pallas_call(
        _kvar_body,
        grid=(NBLK_ROWS,),
        in_specs=[pl.BlockSpec((BM, Fo), lambda g: (g, 0)),
                  pl.BlockSpec((8, Fo), lambda g: (0, 0))],
        out_specs=pl.BlockSpec((8, Fo), lambda g: (0, 0)),
        out_shape=jax.ShapeDtypeStruct((8, Fo), jnp.float32),
    )(pre, sums)


def _k3(s, h, dis, b):
    Fo = s.shape[1]
    return pl.pallas_call(
        _k3_body,
        grid=(NBLK_ROWS,),
        in_specs=[pl.BlockSpec((BM, Fo), lambda g: (g, 0)),
                  pl.BlockSpec((BM, Fo), lambda g: (g, 0)),
                  pl.BlockSpec((BM, 1), lambda g: (g, 0)),
                  pl.BlockSpec((1, Fo), lambda g: (0, 0))],
        out_specs=[pl.BlockSpec((BM, Fo), lambda g: (g, 0)),
                   pl.BlockSpec((8, Fo), lambda g: (0, 0))],
        out_shape=[jax.ShapeDtypeStruct((NP, Fo), jnp.float32),
                   jax.ShapeDtypeStruct((8, Fo), jnp.float32)],
    )(s, h, dis, b)


def _head_body(pre_ref, sums_ref, vsums_ref, gam_ref, bet_ref,
               w1_ref, b1_ref, w2_ref, b2_ref, out_ref):
    t = _lrelu(_bn(pre_ref[...], sums_ref, vsums_ref, gam_ref, bet_ref))
    u = _lrelu(_dot(t, w1_ref[...]) + b1_ref[0, :][None, :])
    out_ref[...] = _dot(u, w2_ref[...]) + b2_ref[0, :][None, :]


def _head(pre, sums, vsums, gam, bet, w1, b1, w2, b2):
    Fi = pre.shape[1]
    F1 = w1.shape[1]
    F2 = w2.shape[1]
    return pl.pallas_call(
        _head_body,
        grid=(NBLK_ROWS,),
        in_specs=[pl.BlockSpec((BM, Fi), lambda g: (g, 0)),
                  pl.BlockSpec((8, Fi), lambda g: (0, 0)),
                  pl.BlockSpec((8, Fi), lambda g: (0, 0)),
                  pl.BlockSpec((1, Fi), lambda g: (0, 0)),
                  pl.BlockSpec((1, Fi), lambda g: (0, 0)),
                  pl.BlockSpec((Fi, F1), lambda g: (0, 0)),
                  pl.BlockSpec((1, F1), lambda g: (0, 0)),
                  pl.BlockSpec((F1, F2), lambda g: (0, 0)),
                  pl.BlockSpec((1, F2), lambda g: (0, 0))],
        out_specs=pl.BlockSpec((BM, F2), lambda g: (g, 0)),
        out_shape=jax.ShapeDtypeStruct((NP, F2), jnp.float32),
    )(pre, sums, vsums, gam, bet, w1, b1, w2, b2)


def kernel(x, edge_index, params):
    E = edge_index.shape[1]
    pad = EP - E
    sent = jnp.concatenate(
        [jnp.full((1, pad), SENT, jnp.int32), jnp.zeros((1, pad), jnp.int32)],
        axis=0)
    eix = jnp.concatenate([edge_index.astype(jnp.int32), sent], axis=1)
    src = eix[0]
    dst = eix[1]

    bsrc, bdstl, cnts = _make_preprocess()(src, dst)
    cnt = _make_degrees()(bsrc, bdstl, cnts)
    dis = _dis_kernel(cnt.reshape(NP, 1))

    xp = jnp.pad(x, ((0, NP - x.shape[0]), (0, 0)))
    h = _k1_first(xp, params["W"][0], dis)
    pre = None
    sums = None
    for i in range(12):
        Fo = h.shape[1]
        zF = jnp.zeros((7 if Fo >= 512 else 49, Fo), jnp.float32)
        s = _make_segsum(Fo)(h, bsrc, bdstl, cnts, zF)
        pre, sums = _k3(s, h, dis, params["b"][i].reshape(1, -1))
        vsums = _kvar(pre, sums)
        if i < 11:
            h = _k1_mid(pre, sums, vsums,
                        params["g"][i].reshape(1, -1),
                        params["be"][i].reshape(1, -1),
                        params["W"][i + 1], dis)
    out = _head(pre, sums, vsums,
                params["g"][11].reshape(1, -1),
                params["be"][11].reshape(1, -1),
                params["lW1"], params["lb1"].reshape(1, -1),
                params["lW2"], params["lb2"].reshape(1, -1))
    return out[:N]
